# baseline (speedup 1.0000x reference)
import functools

import jax
import jax.numpy as jnp
from jax import lax
from jax.experimental import pallas as pl
from jax.experimental.pallas import tpu as pltpu
from jax.experimental.pallas import tpu_sc as plsc

N_E = 8192
D = 32
N_ROWS = 8192
BLK = 128
N_BLK = N_ROWS // BLK

_SC_NC = 2
_SC_NS = 16
_SC_NW = _SC_NC * _SC_NS
_ROWS_PER_W = N_ROWS // _SC_NW


def _argmin_body(x_ref, e_ref, xn_ref, en_ref, m_ref, idx_ref):
    dots_t = lax.dot_general(
        e_ref[...], x_ref[...], (((1,), (1,)), ((), ())),
        preferred_element_type=jnp.float32)
    d2 = xn_ref[...] + en_ref[...] - 2.0 * dots_t
    d = jnp.sqrt(jnp.maximum(d2, 0.0))
    m = jnp.min(d, axis=0, keepdims=True)
    m_ref[...] = m
    rows = lax.broadcasted_iota(jnp.int32, (N_E, BLK), 0)
    idx_ref[...] = jnp.min(jnp.where(d == m, rows, N_E), axis=0)


def _nearest_codes(flat_x, embeddings, x_norm, e_norm):
    return pl.pallas_call(
        _argmin_body,
        grid=(N_BLK,),
        in_specs=[
            pl.BlockSpec((BLK, D), lambda i: (i, 0)),
            pl.BlockSpec((N_E, D), lambda i: (0, 0)),
            pl.BlockSpec((1, BLK), lambda i: (0, i)),
            pl.BlockSpec((N_E, 1), lambda i: (0, 0)),
        ],
        out_specs=[
            pl.BlockSpec((1, BLK), lambda i: (0, i)),
            pl.BlockSpec((BLK,), lambda i: (i,)),
        ],
        out_shape=[
            jax.ShapeDtypeStruct((1, N_ROWS), jnp.float32),
            jax.ShapeDtypeStruct((N_ROWS,), jnp.int32),
        ],
    )(flat_x, embeddings, x_norm, e_norm)


@functools.cache
def _build_sc_gather():
    @functools.partial(
        pl.kernel,
        mesh=plsc.VectorSubcoreMesh(core_axis_name="c", subcore_axis_name="s"),
        out_type=jax.ShapeDtypeStruct((N_ROWS, D), jnp.float32),
        scratch_types=[
            pltpu.VMEM((_ROWS_PER_W,), jnp.int32),
            pltpu.VMEM((_ROWS_PER_W, D), jnp.float32),
            pltpu.SemaphoreType.DMA,
        ],
        compiler_params=pltpu.CompilerParams(use_tc_tiling_on_sc=False),
    )
    def _sc_gather(table_hbm, idx_hbm, out_hbm, idx_v, rows_v, sem):
        wid = lax.axis_index("s") * _SC_NC + lax.axis_index("c")
        base = wid * _ROWS_PER_W
        pltpu.sync_copy(idx_hbm.at[pl.ds(base, _ROWS_PER_W)], idx_v)
        pltpu.async_copy(table_hbm.at[idx_v], rows_v, sem).wait()
        pltpu.sync_copy(rows_v, out_hbm.at[pl.ds(base, _ROWS_PER_W)])

    return _sc_gather


def kernel(inputs, embeddings):
    input_shape = inputs.shape
    flat_x = inputs.reshape(-1, D)
    x_norm = jnp.sum(inputs ** 2, axis=2).reshape(1, -1)
    e_norm = jnp.sum(embeddings ** 2, axis=1, keepdims=True)
    flat_r = flat_x.astype(jnp.bfloat16).astype(jnp.float32)
    m, idx = _nearest_codes(flat_r, embeddings, x_norm, e_norm)
    quantized = _build_sc_gather()(embeddings, idx).reshape(input_shape)
    loss = jnp.sum(m * m) * jnp.float32(1.0 / (N_ROWS * D))
    return (quantized, loss, idx)


# speedup vs baseline: 1.0238x; 1.0238x over previous
import functools

import jax
import jax.numpy as jnp
from jax import lax
from jax.experimental import pallas as pl
from jax.experimental.pallas import tpu as pltpu
from jax.experimental.pallas import tpu_sc as plsc

N_E = 8192
D = 32
N_ROWS = 8192
BLK = 256
N_BLK = N_ROWS // BLK

_SC_NC = 2
_SC_NS = 16
_SC_NW = _SC_NC * _SC_NS
_ROWS_PER_W = N_ROWS // _SC_NW


def _argmin_body(x_ref, e_ref, xn_ref, en_ref, m_ref, idx_ref):
    dots_t = lax.dot_general(
        e_ref[...], x_ref[...], (((1,), (1,)), ((), ())),
        preferred_element_type=jnp.float32)
    d2 = xn_ref[...] + en_ref[...] - 2.0 * dots_t
    d = jnp.sqrt(jnp.maximum(d2, 0.0))
    m = jnp.min(d, axis=0, keepdims=True)
    m_ref[...] = m
    rows = lax.broadcasted_iota(jnp.int32, (N_E, BLK), 0)
    idx_ref[...] = jnp.min(jnp.where(d == m, rows, N_E), axis=0)


def _nearest_codes(flat_x, embeddings, x_norm, e_norm):
    return pl.pallas_call(
        _argmin_body,
        grid=(N_BLK,),
        in_specs=[
            pl.BlockSpec((BLK, D), lambda i: (i, 0)),
            pl.BlockSpec((N_E, D), lambda i: (0, 0)),
            pl.BlockSpec((1, BLK), lambda i: (0, i)),
            pl.BlockSpec((N_E, 1), lambda i: (0, 0)),
        ],
        out_specs=[
            pl.BlockSpec((1, BLK), lambda i: (0, i)),
            pl.BlockSpec((BLK,), lambda i: (i,)),
        ],
        out_shape=[
            jax.ShapeDtypeStruct((1, N_ROWS), jnp.float32),
            jax.ShapeDtypeStruct((N_ROWS,), jnp.int32),
        ],
    )(flat_x, embeddings, x_norm, e_norm)


@functools.cache
def _build_sc_gather():
    @functools.partial(
        pl.kernel,
        mesh=plsc.VectorSubcoreMesh(core_axis_name="c", subcore_axis_name="s"),
        out_type=jax.ShapeDtypeStruct((N_ROWS, D), jnp.float32),
        scratch_types=[
            pltpu.VMEM((_ROWS_PER_W,), jnp.int32),
            pltpu.VMEM((_ROWS_PER_W, D), jnp.float32),
            pltpu.SemaphoreType.DMA,
        ],
        compiler_params=pltpu.CompilerParams(use_tc_tiling_on_sc=False),
    )
    def _sc_gather(table_hbm, idx_hbm, out_hbm, idx_v, rows_v, sem):
        wid = lax.axis_index("s") * _SC_NC + lax.axis_index("c")
        base = wid * _ROWS_PER_W
        pltpu.sync_copy(idx_hbm.at[pl.ds(base, _ROWS_PER_W)], idx_v)
        pltpu.async_copy(table_hbm.at[idx_v], rows_v, sem).wait()
        pltpu.sync_copy(rows_v, out_hbm.at[pl.ds(base, _ROWS_PER_W)])

    return _sc_gather


def kernel(inputs, embeddings):
    input_shape = inputs.shape
    flat_x = inputs.reshape(-1, D)
    x_norm = jnp.sum(inputs ** 2, axis=2).reshape(1, -1)
    e_norm = jnp.sum(embeddings ** 2, axis=1, keepdims=True)
    flat_r = flat_x.astype(jnp.bfloat16).astype(jnp.float32)
    m, idx = _nearest_codes(flat_r, embeddings, x_norm, e_norm)
    quantized = _build_sc_gather()(embeddings, idx).reshape(input_shape)
    loss = jnp.sum(m * m) * jnp.float32(1.0 / (N_ROWS * D))
    return (quantized, loss, idx)
